# Initial kernel scaffold; baseline (speedup 1.0000x reference)
import functools

import jax
import jax.numpy as jnp
from jax.experimental import pallas as pl

B, S = 8, 576
N = B * S
IN_DIM, HID, K_DIM, Z_DIM = 768, 2048, 1024, 64
M_BLK = 512

_HI = jax.lax.Precision.HIGHEST


def _first_argmin(scores, cols):
    m = jnp.min(scores, axis=1, keepdims=True)
    return jnp.min(jnp.where(scores == m, cols, K_DIM), axis=1)


def _vqvae_block(x_ref, w1_ref, b1_ref, w2_ref, b2_ref, e_ref, w3_ref,
                 b3_ref, w4_ref, b4_ref, recon_ref, zenc_ref, zemb_ref):
    h = jnp.maximum(jnp.dot(x_ref[...], w1_ref[...], precision=_HI)
                    + b1_ref[...], 0.0)
    z = jnp.dot(h, w2_ref[...], precision=_HI) + b2_ref[...]
    zenc_ref[...] = z

    embd = e_ref[...]
    se = jnp.sum(embd * embd, axis=1)
    scores = se[None, :] - 2.0 * jnp.dot(z, embd.T, precision=_HI)

    cols = jax.lax.broadcasted_iota(jnp.int32, (M_BLK, K_DIM), 1)
    idx1 = _first_argmin(scores, cols)
    mask1 = cols == idx1[:, None]
    idx2 = _first_argmin(jnp.where(mask1, jnp.inf, scores), cols)
    mask2 = cols == idx2[:, None]

    e1 = jnp.dot(mask1.astype(jnp.float32), embd, precision=_HI)
    e2 = jnp.dot(mask2.astype(jnp.float32), embd, precision=_HI)
    d1 = jnp.sum((z - e1) ** 2, axis=1)
    d2 = jnp.sum((z - e2) ** 2, axis=1)
    swap = (d2 < d1) | ((d2 == d1) & (idx2 < idx1))
    e_sel = jnp.where(swap[:, None], e2, e1)
    zemb_ref[...] = e_sel

    h2 = jnp.dot(e_sel, w3_ref[...], precision=_HI) + b3_ref[...]
    h2 = jnp.where(h2 > 0, h2, 0.1 * h2)
    recon_ref[...] = jnp.dot(h2, w4_ref[...], precision=_HI) + b4_ref[...]


@functools.partial(jax.jit, static_argnames=("interpret",))
def _run(X, W1, b1, W2, b2, embd, W3, b3, W4, b4, interpret=False):
    x2 = X.reshape(N, IN_DIM)
    grid = (N // M_BLK,)
    full = lambda shape: pl.BlockSpec(shape, lambda i: (0, 0))
    recon, zenc, zemb = pl.pallas_call(
        _vqvae_block,
        grid=grid,
        in_specs=[
            pl.BlockSpec((M_BLK, IN_DIM), lambda i: (i, 0)),
            full((IN_DIM, HID)),
            full((1, HID)),
            full((HID, Z_DIM)),
            full((1, Z_DIM)),
            full((K_DIM, Z_DIM)),
            full((Z_DIM, HID)),
            full((1, HID)),
            full((HID, IN_DIM)),
            full((1, IN_DIM)),
        ],
        out_specs=[
            pl.BlockSpec((M_BLK, IN_DIM), lambda i: (i, 0)),
            pl.BlockSpec((M_BLK, Z_DIM), lambda i: (i, 0)),
            pl.BlockSpec((M_BLK, Z_DIM), lambda i: (i, 0)),
        ],
        out_shape=[
            jax.ShapeDtypeStruct((N, IN_DIM), jnp.float32),
            jax.ShapeDtypeStruct((N, Z_DIM), jnp.float32),
            jax.ShapeDtypeStruct((N, Z_DIM), jnp.float32),
        ],
        interpret=interpret,
    )(x2, W1, b1.reshape(1, HID), W2, b2.reshape(1, Z_DIM), embd,
      W3, b3.reshape(1, HID), W4, b4.reshape(1, IN_DIM))
    return (recon.reshape(B, S, IN_DIM), zenc.reshape(B, S, Z_DIM),
            zemb.reshape(B, S, Z_DIM))


def kernel(X, W1, b1, W2, b2, embd, W3, b3, W4, b4):
    return _run(X, W1, b1, W2, b2, embd, W3, b3, W4, b4)


# speedup vs baseline: 3.0607x; 3.0607x over previous
import functools

import jax
import jax.numpy as jnp
from jax.experimental import pallas as pl

B, S = 8, 576
N = B * S
IN_DIM, HID, K_DIM, Z_DIM = 768, 2048, 1024, 64
M_BLK = 128
KC = 128
NKC = K_DIM // KC

_HI = jax.lax.Precision.HIGHEST
_DEF = jax.lax.Precision.DEFAULT


def _vq_block(x_ref, w1_ref, b1_ref, w2_ref, b2_ref, e_ref, et_ref, w3_ref,
              b3_ref, w4_ref, b4_ref, recon_ref, zenc_ref, zemb_ref):
    h = jnp.maximum(jnp.dot(x_ref[...], w1_ref[...], precision=_DEF)
                    + b1_ref[...], 0.0)
    z = jnp.dot(h, w2_ref[...], precision=_DEF) + b2_ref[...]
    zenc_ref[...] = z

    et = et_ref[...]
    lane = jax.lax.broadcasted_iota(jnp.int32, (M_BLK, KC), 1)
    sc, run_min = [], None
    for c in range(NKC):
        etc = et[:, c * KC:(c + 1) * KC]
        se_c = jnp.sum(etc * etc, axis=0, keepdims=True)
        s = se_c - 2.0 * jnp.dot(z, etc, precision=_HI)
        sc.append(s)
        run_min = s if run_min is None else jnp.minimum(run_min, s)
    gmin = jnp.min(run_min, axis=1, keepdims=True)

    def argmin_from(chunks, gm):
        cand = None
        for c in range(NKC):
            cc = jnp.where(chunks[c] == gm, lane + c * KC, K_DIM)
            cand = cc if cand is None else jnp.minimum(cand, cc)
        return jnp.min(cand, axis=1, keepdims=True)

    idx1 = argmin_from(sc, gmin)

    sc2, run_min2 = [], None
    for c in range(NKC):
        s2 = jnp.where(lane + c * KC == idx1, jnp.inf, sc[c])
        sc2.append(s2)
        run_min2 = s2 if run_min2 is None else jnp.minimum(run_min2, s2)
    gmin2 = jnp.min(run_min2, axis=1, keepdims=True)
    idx2 = argmin_from(sc2, gmin2)

    embd = e_ref[...]
    e1 = e2 = None
    for c in range(NKC):
        col = lane + c * KC
        ec = embd[c * KC:(c + 1) * KC, :]
        p1 = jnp.dot((col == idx1).astype(jnp.float32), ec, precision=_HI)
        p2 = jnp.dot((col == idx2).astype(jnp.float32), ec, precision=_HI)
        e1 = p1 if e1 is None else e1 + p1
        e2 = p2 if e2 is None else e2 + p2

    d1 = jnp.sum((z - e1) ** 2, axis=1, keepdims=True)
    d2 = jnp.sum((z - e2) ** 2, axis=1, keepdims=True)
    swap = (d2 < d1) | ((d2 == d1) & (idx2 < idx1))
    e_sel = jnp.where(swap, e2, e1)
    zemb_ref[...] = e_sel

    h2 = jnp.dot(e_sel, w3_ref[...], precision=_DEF) + b3_ref[...]
    h2 = jnp.where(h2 > 0, h2, 0.1 * h2)
    recon_ref[...] = jnp.dot(h2, w4_ref[...], precision=_DEF) + b4_ref[...]


@functools.partial(jax.jit, static_argnames=("interpret",))
def _run(X, W1, b1, W2, b2, embd, W3, b3, W4, b4, interpret=False):
    x2 = X.reshape(N, IN_DIM)
    grid = (N // M_BLK,)
    full = lambda shape: pl.BlockSpec(shape, lambda i: (0, 0))
    recon, zenc, zemb = pl.pallas_call(
        _vq_block,
        grid=grid,
        in_specs=[
            pl.BlockSpec((M_BLK, IN_DIM), lambda i: (i, 0)),
            full((IN_DIM, HID)),
            full((1, HID)),
            full((HID, Z_DIM)),
            full((1, Z_DIM)),
            full((K_DIM, Z_DIM)),
            full((Z_DIM, K_DIM)),
            full((Z_DIM, HID)),
            full((1, HID)),
            full((HID, IN_DIM)),
            full((1, IN_DIM)),
        ],
        out_specs=[
            pl.BlockSpec((M_BLK, IN_DIM), lambda i: (i, 0)),
            pl.BlockSpec((M_BLK, Z_DIM), lambda i: (i, 0)),
            pl.BlockSpec((M_BLK, Z_DIM), lambda i: (i, 0)),
        ],
        out_shape=[
            jax.ShapeDtypeStruct((N, IN_DIM), jnp.float32),
            jax.ShapeDtypeStruct((N, Z_DIM), jnp.float32),
            jax.ShapeDtypeStruct((N, Z_DIM), jnp.float32),
        ],
        interpret=interpret,
    )(x2, W1, b1.reshape(1, HID), W2, b2.reshape(1, Z_DIM), embd, embd.T,
      W3, b3.reshape(1, HID), W4, b4.reshape(1, IN_DIM))
    return (recon.reshape(B, S, IN_DIM), zenc.reshape(B, S, Z_DIM),
            zemb.reshape(B, S, Z_DIM))


def kernel(X, W1, b1, W2, b2, embd, W3, b3, W4, b4):
    return _run(X, W1, b1, W2, b2, embd, W3, b3, W4, b4)


# speedup vs baseline: 3.1458x; 1.0278x over previous
import functools

import jax
import jax.numpy as jnp
from jax.experimental import pallas as pl

B, S = 8, 576
N = B * S
IN_DIM, HID, K_DIM, Z_DIM = 768, 2048, 1024, 64
M_BLK = 128
KC = 128
NKC = K_DIM // KC

BF = jnp.bfloat16
F32 = jnp.float32


def _dotf(a, b):
    return jnp.dot(a, b, preferred_element_type=F32)


def _vq_block(x_ref, w1_ref, b1_ref, w2_ref, b2_ref,
              eh_ref, em_ref, el_ref, eth_ref, etm_ref, etl_ref,
              w3_ref, b3_ref, w4_ref, b4_ref,
              recon_ref, zenc_ref, zemb_ref):
    h = jnp.maximum(_dotf(x_ref[...], w1_ref[...]) + b1_ref[...], 0.0)
    z = _dotf(h.astype(BF), w2_ref[...]) + b2_ref[...]
    zenc_ref[...] = z

    zh = z.astype(BF)
    zl = (z - zh.astype(F32)).astype(BF)

    eth, etm, etl = eth_ref[...], etm_ref[...], etl_ref[...]
    lane = jax.lax.broadcasted_iota(jnp.int32, (M_BLK, KC), 1)
    sc, run_min = [], None
    for c in range(NKC):
        sl = slice(c * KC, (c + 1) * KC)
        etc = (eth[:, sl].astype(F32) + etm[:, sl].astype(F32)
               + etl[:, sl].astype(F32))
        se_c = jnp.sum(etc * etc, axis=0, keepdims=True)
        zdote = (_dotf(zh, eth[:, sl]) + (_dotf(zh, etm[:, sl])
                                          + _dotf(zl, eth[:, sl])))
        s = se_c - 2.0 * zdote
        sc.append(s)
        run_min = s if run_min is None else jnp.minimum(run_min, s)
    gmin = jnp.min(run_min, axis=1, keepdims=True)

    def argmin_from(chunks, gm):
        cand = None
        for c in range(NKC):
            cc = jnp.where(chunks[c] == gm, lane + c * KC, K_DIM)
            cand = cc if cand is None else jnp.minimum(cand, cc)
        return jnp.min(cand, axis=1, keepdims=True)

    idx1 = argmin_from(sc, gmin)

    sc2, run_min2 = [], None
    for c in range(NKC):
        s2 = jnp.where(lane + c * KC == idx1, jnp.inf, sc[c])
        sc2.append(s2)
        run_min2 = s2 if run_min2 is None else jnp.minimum(run_min2, s2)
    gmin2 = jnp.min(run_min2, axis=1, keepdims=True)
    idx2 = argmin_from(sc2, gmin2)

    eh, em, el = eh_ref[...], em_ref[...], el_ref[...]
    e1 = e2 = None
    for c in range(NKC):
        col = lane + c * KC
        sl = slice(c * KC, (c + 1) * KC)
        oh1 = (col == idx1).astype(BF)
        oh2 = (col == idx2).astype(BF)
        p1 = (_dotf(oh1, eh[sl]) + (_dotf(oh1, em[sl]) + _dotf(oh1, el[sl])))
        p2 = (_dotf(oh2, eh[sl]) + (_dotf(oh2, em[sl]) + _dotf(oh2, el[sl])))
        e1 = p1 if e1 is None else e1 + p1
        e2 = p2 if e2 is None else e2 + p2

    d1 = jnp.sum((z - e1) ** 2, axis=1, keepdims=True)
    d2 = jnp.sum((z - e2) ** 2, axis=1, keepdims=True)
    swap = (d2 < d1) | ((d2 == d1) & (idx2 < idx1))
    e_sel = jnp.where(swap, e2, e1)
    zemb_ref[...] = e_sel

    h2 = _dotf(e_sel.astype(BF), w3_ref[...]) + b3_ref[...]
    h2 = jnp.where(h2 > 0, h2, 0.1 * h2)
    recon_ref[...] = _dotf(h2.astype(BF), w4_ref[...]) + b4_ref[...]


def _split3(a):
    hi = a.astype(BF)
    r = a - hi.astype(F32)
    mid = r.astype(BF)
    lo = (r - mid.astype(F32)).astype(BF)
    return hi, mid, lo


@functools.partial(jax.jit, static_argnames=("interpret",))
def _run(X, W1, b1, W2, b2, embd, W3, b3, W4, b4, interpret=False):
    x2 = X.reshape(N, IN_DIM).astype(BF)
    eh, em, el = _split3(embd)
    grid = (N // M_BLK,)
    full = lambda shape: pl.BlockSpec(shape, lambda i: (0, 0))
    recon, zenc, zemb = pl.pallas_call(
        _vq_block,
        grid=grid,
        in_specs=[
            pl.BlockSpec((M_BLK, IN_DIM), lambda i: (i, 0)),
            full((IN_DIM, HID)),
            full((1, HID)),
            full((HID, Z_DIM)),
            full((1, Z_DIM)),
            full((K_DIM, Z_DIM)),
            full((K_DIM, Z_DIM)),
            full((K_DIM, Z_DIM)),
            full((Z_DIM, K_DIM)),
            full((Z_DIM, K_DIM)),
            full((Z_DIM, K_DIM)),
            full((Z_DIM, HID)),
            full((1, HID)),
            full((HID, IN_DIM)),
            full((1, IN_DIM)),
        ],
        out_specs=[
            pl.BlockSpec((M_BLK, IN_DIM), lambda i: (i, 0)),
            pl.BlockSpec((M_BLK, Z_DIM), lambda i: (i, 0)),
            pl.BlockSpec((M_BLK, Z_DIM), lambda i: (i, 0)),
        ],
        out_shape=[
            jax.ShapeDtypeStruct((N, IN_DIM), F32),
            jax.ShapeDtypeStruct((N, Z_DIM), F32),
            jax.ShapeDtypeStruct((N, Z_DIM), F32),
        ],
        interpret=interpret,
    )(x2, W1.astype(BF), b1.reshape(1, HID), W2.astype(BF),
      b2.reshape(1, Z_DIM), eh, em, el, eh.T, em.T, el.T,
      W3.astype(BF), b3.reshape(1, HID), W4.astype(BF), b4.reshape(1, IN_DIM))
    return (recon.reshape(B, S, IN_DIM), zenc.reshape(B, S, Z_DIM),
            zemb.reshape(B, S, Z_DIM))


def kernel(X, W1, b1, W2, b2, embd, W3, b3, W4, b4):
    return _run(X, W1, b1, W2, b2, embd, W3, b3, W4, b4)
